# baseline (speedup 1.0000x reference)
import functools

import jax
import jax.numpy as jnp
from jax import lax
from jax.experimental import pallas as pl
from jax.experimental.pallas import tpu as pltpu
from jax.experimental.pallas import tpu_sc as plsc

_CH = 24
_HW = 1024
_NV = _HW // 16


def _sc_body(x_hbm, o_hbm, in0, in1, ob0, ob1, acc, keys, idxb, maskb,
             in_sems, out_sems, *, k, c_total):
    nch = c_total // _CH
    npair = nch // 2
    b = lax.axis_index("s") * 2 + lax.axis_index("c")
    sample = c_total * _HW
    chunk = _CH * _HW

    ins = (in0, in1)
    obs = (ob0, ob1)

    def in_copy(q, slot):
        return pltpu.make_async_copy(
            x_hbm.at[pl.ds(b * sample + q * chunk, chunk)], ins[slot],
            in_sems.at[slot],
        )

    def out_copy(q, slot):
        return pltpu.make_async_copy(
            obs[slot], o_hbm.at[pl.ds(b * sample + q * chunk, chunk)],
            out_sems.at[slot],
        )

    zf = jnp.zeros((16,), jnp.float32)
    iota = lax.broadcasted_iota(jnp.int32, (16,), 0)
    for v in range(_NV):
        acc[pl.ds(16 * v, 16)] = zf
        idxb[pl.ds(16 * v, 16)] = iota + jnp.full((16,), 16 * v, jnp.int32)

    in_copy(0, 0).start()
    in_copy(1, 1).start()

    def accum_chunk(buf):
        def vbody(v, _):
            off = 16 * v
            a = acc[pl.ds(off, 16)]
            for cc in range(_CH):
                a = a + buf[pl.ds(cc * _HW + off, 16)]
            acc[pl.ds(off, 16)] = a
            return 0

        lax.fori_loop(0, _NV, vbody, 0)

    def p1body(p, _):
        q = 2 * p
        for slot in (0, 1):
            in_copy(q + slot, slot).wait()
            accum_chunk(ins[slot])

            @pl.when(p < npair - 1)
            def _():
                in_copy(q + slot + 2, slot).start()

        return 0

    lax.fori_loop(0, npair, p1body, 0)

    c31 = jnp.full((16,), 31, jnp.uint32)
    c1 = jnp.full((16,), 1, jnp.uint32)
    call1 = jnp.full((16,), 0xFFFFFFFF, jnp.uint32)
    csign = jnp.full((16,), 0x80000000, jnp.uint32)

    for v in range(_NV):
        sl = pl.ds(16 * v, 16)
        bits = lax.bitcast_convert_type(acc[sl], jnp.uint32)
        neg = (bits >> c31) == c1
        keys[sl] = bits ^ jnp.where(neg, call1, csign)

    kk = jnp.int32(k)
    ones_i = jnp.full((16,), 1, jnp.int32)
    zeros_i = jnp.full((16,), 0, jnp.int32)

    def splat(x):
        return lax.broadcast_in_dim(x, (16,), ())

    def count_where(pred):
        cvec = zeros_i
        for v in range(_NV):
            sl = pl.ds(16 * v, 16)
            cvec = cvec + jnp.where(pred(keys[sl], idxb[sl]), ones_i, zeros_i)
        return jnp.sum(cvec)

    def vround(i, t):
        shift = jnp.uint32(31) - i.astype(jnp.uint32)
        cand = t | (jnp.uint32(1) << shift)
        cand_v = splat(cand)
        cnt = count_where(lambda kv, idx: kv >= cand_v)
        return jnp.where(cnt >= kk, cand, t)

    t = lax.fori_loop(0, 32, vround, jnp.uint32(0))
    t_v = splat(t)

    extra = kk - count_where(lambda kv, idx: kv > t_v)

    def iround(i, j):
        shift = jnp.int32(10) - i
        cand = j | (jnp.int32(1) << shift)
        cand_v = splat(cand)
        cnt = count_where(lambda kv, idx: (kv == t_v) & (idx < cand_v))
        return jnp.where(cnt <= extra, cand, j)

    jmax_v = splat(lax.fori_loop(0, 11, iround, jnp.int32(0)))

    one_f = jnp.full((16,), 1.0, jnp.float32)
    zero_f = jnp.full((16,), 0.0, jnp.float32)
    for v in range(_NV):
        sl = pl.ds(16 * v, 16)
        kv = keys[sl]
        keep = (kv > t_v) | ((kv == t_v) & (idxb[sl] < jmax_v))
        maskb[sl] = jnp.where(keep, one_f, zero_f)

    in_copy(0, 0).start()
    in_copy(1, 1).start()

    def mul_chunk(inb, ob):
        def vbody(v, _):
            off = 16 * v
            m = maskb[pl.ds(off, 16)]
            for cc in range(_CH):
                sl = pl.ds(cc * _HW + off, 16)
                ob[sl] = inb[sl] * m
            return 0

        lax.fori_loop(0, _NV, vbody, 0)

    def p2body(p, _):
        q = 2 * p
        for slot in (0, 1):
            in_copy(q + slot, slot).wait()

            @pl.when(p > 0)
            def _():
                out_copy(q + slot - 2, slot).wait()

            mul_chunk(ins[slot], obs[slot])
            out_copy(q + slot, slot).start()

            @pl.when(p < npair - 1)
            def _():
                in_copy(q + slot + 2, slot).start()

        return 0

    lax.fori_loop(0, npair, p2body, 0)

    out_copy(nch - 2, 0).wait()
    out_copy(nch - 1, 1).wait()


def kernel(x):
    B, C, H, W = x.shape
    HW = H * W
    k = int(HW * 0.5)
    assert HW == _HW and C % (2 * _CH) == 0
    xf = x.reshape(B * C * HW)

    mesh = plsc.VectorSubcoreMesh(core_axis_name="c", subcore_axis_name="s")

    body = functools.partial(_sc_body, k=k, c_total=C)
    f = pl.kernel(
        body,
        out_type=jax.ShapeDtypeStruct((B * C * HW,), jnp.float32),
        mesh=mesh,
        compiler_params=pltpu.CompilerParams(needs_layout_passes=False),
        scratch_types=[
            pltpu.VMEM((_CH * _HW,), jnp.float32),
            pltpu.VMEM((_CH * _HW,), jnp.float32),
            pltpu.VMEM((_CH * _HW,), jnp.float32),
            pltpu.VMEM((_CH * _HW,), jnp.float32),
            pltpu.VMEM((_HW,), jnp.float32),
            pltpu.VMEM((_HW,), jnp.uint32),
            pltpu.VMEM((_HW,), jnp.int32),
            pltpu.VMEM((_HW,), jnp.float32),
            pltpu.SemaphoreType.DMA((2,)),
            pltpu.SemaphoreType.DMA((2,)),
        ],
    )
    out = f(xf)
    return out.reshape(B, C, H, W)


# speedup vs baseline: 2.1140x; 2.1140x over previous
import functools

import jax
import jax.numpy as jnp
from jax import lax
from jax.experimental import pallas as pl
from jax.experimental.pallas import tpu as pltpu
from jax.experimental.pallas import tpu_sc as plsc

_CH = 24
_HW = 1024
_NV = _HW // 16


def _sc_body(x_hbm, o_hbm, in0, in1, ob0, ob1, acc, keys, idxb, maskb,
             in_sems, out_sems, *, k, c_total):
    nch = c_total // _CH
    npair = nch // 2
    b = lax.axis_index("s") * 2 + lax.axis_index("c")

    ins = (in0, in1)
    obs = (ob0, ob1)

    def in_copy(q, slot):
        return pltpu.make_async_copy(
            x_hbm.at[b, pl.ds(q * _CH, _CH)], ins[slot], in_sems.at[slot]
        )

    def out_copy(q, slot):
        return pltpu.make_async_copy(
            obs[slot], o_hbm.at[b, pl.ds(q * _CH, _CH)], out_sems.at[slot]
        )

    zf = jnp.zeros((16,), jnp.float32)
    iota = lax.broadcasted_iota(jnp.int32, (16,), 0)
    for v in range(_NV):
        acc[pl.ds(16 * v, 16)] = zf
        idxb[pl.ds(16 * v, 16)] = iota + jnp.full((16,), 16 * v, jnp.int32)

    in_copy(0, 0).start()
    in_copy(1, 1).start()

    def accum_chunk(buf):
        def vbody(sg, _):
            for l in range(8):
                off = pl.ds(128 * sg + 16 * l, 16)
                a = acc[off]
                for cc in range(_CH):
                    a = a + buf[cc, sg, pl.ds(16 * l, 16)]
                acc[off] = a
            return 0

        lax.fori_loop(0, 8, vbody, 0)

    def p1body(p, _):
        q = 2 * p
        for slot in (0, 1):
            in_copy(q + slot, slot).wait()
            accum_chunk(ins[slot])

            @pl.when(p < npair - 1)
            def _():
                in_copy(q + slot + 2, slot).start()

        return 0

    lax.fori_loop(0, npair, p1body, 0)

    c31 = jnp.full((16,), 31, jnp.uint32)
    c1 = jnp.full((16,), 1, jnp.uint32)
    call1 = jnp.full((16,), 0xFFFFFFFF, jnp.uint32)
    csign = jnp.full((16,), 0x80000000, jnp.uint32)

    for v in range(_NV):
        sl = pl.ds(16 * v, 16)
        bits = lax.bitcast_convert_type(acc[sl], jnp.uint32)
        neg = (bits >> c31) == c1
        keys[sl] = bits ^ jnp.where(neg, call1, csign)

    kk = jnp.int32(k)
    ones_i = jnp.full((16,), 1, jnp.int32)
    zeros_i = jnp.full((16,), 0, jnp.int32)

    def splat(x):
        return lax.broadcast_in_dim(x, (16,), ())

    def count_where(pred):
        cvec = zeros_i
        for v in range(_NV):
            sl = pl.ds(16 * v, 16)
            cvec = cvec + jnp.where(pred(keys[sl], idxb[sl]), ones_i, zeros_i)
        return jnp.sum(cvec)

    def vround(i, t):
        shift = jnp.uint32(31) - i.astype(jnp.uint32)
        cand = t | (jnp.uint32(1) << shift)
        cand_v = splat(cand)
        cnt = count_where(lambda kv, idx: kv >= cand_v)
        return jnp.where(cnt >= kk, cand, t)

    t = lax.fori_loop(0, 32, vround, jnp.uint32(0))
    t_v = splat(t)

    extra = kk - count_where(lambda kv, idx: kv > t_v)

    def iround(i, j):
        shift = jnp.int32(10) - i
        cand = j | (jnp.int32(1) << shift)
        cand_v = splat(cand)
        cnt = count_where(lambda kv, idx: (kv == t_v) & (idx < cand_v))
        return jnp.where(cnt <= extra, cand, j)

    jmax_v = splat(lax.fori_loop(0, 11, iround, jnp.int32(0)))

    one_f = jnp.full((16,), 1.0, jnp.float32)
    zero_f = jnp.full((16,), 0.0, jnp.float32)
    for v in range(_NV):
        sl = pl.ds(16 * v, 16)
        kv = keys[sl]
        keep = (kv > t_v) | ((kv == t_v) & (idxb[sl] < jmax_v))
        maskb[sl] = jnp.where(keep, one_f, zero_f)

    in_copy(0, 0).start()
    in_copy(1, 1).start()

    def mul_chunk(inb, ob):
        def vbody(sg, _):
            for l in range(8):
                m = maskb[pl.ds(128 * sg + 16 * l, 16)]
                for cc in range(_CH):
                    sl = pl.ds(16 * l, 16)
                    ob[cc, sg, sl] = inb[cc, sg, sl] * m
            return 0

        lax.fori_loop(0, 8, vbody, 0)

    def p2body(p, _):
        q = 2 * p
        for slot in (0, 1):
            in_copy(q + slot, slot).wait()

            @pl.when(p > 0)
            def _():
                out_copy(q + slot - 2, slot).wait()

            mul_chunk(ins[slot], obs[slot])
            out_copy(q + slot, slot).start()

            @pl.when(p < npair - 1)
            def _():
                in_copy(q + slot + 2, slot).start()

        return 0

    lax.fori_loop(0, npair, p2body, 0)

    out_copy(nch - 2, 0).wait()
    out_copy(nch - 1, 1).wait()


def kernel(x):
    B, C, H, W = x.shape
    HW = H * W
    k = int(HW * 0.5)
    assert HW == _HW and C % (2 * _CH) == 0
    xr = x.reshape(B, C, 8, 128)

    mesh = plsc.VectorSubcoreMesh(core_axis_name="c", subcore_axis_name="s")

    body = functools.partial(_sc_body, k=k, c_total=C)
    f = pl.kernel(
        body,
        out_type=jax.ShapeDtypeStruct((B, C, 8, 128), jnp.float32),
        mesh=mesh,
        compiler_params=pltpu.CompilerParams(needs_layout_passes=False),
        scratch_types=[
            pltpu.VMEM((_CH, 8, 128), jnp.float32),
            pltpu.VMEM((_CH, 8, 128), jnp.float32),
            pltpu.VMEM((_CH, 8, 128), jnp.float32),
            pltpu.VMEM((_CH, 8, 128), jnp.float32),
            pltpu.VMEM((_HW,), jnp.float32),
            pltpu.VMEM((_HW,), jnp.uint32),
            pltpu.VMEM((_HW,), jnp.int32),
            pltpu.VMEM((_HW,), jnp.float32),
            pltpu.SemaphoreType.DMA((2,)),
            pltpu.SemaphoreType.DMA((2,)),
        ],
    )
    out = f(xr)
    return out.reshape(B, C, H, W)


# speedup vs baseline: 2.2692x; 1.0734x over previous
import functools

import jax
import jax.numpy as jnp
from jax import lax
from jax.experimental import pallas as pl
from jax.experimental.pallas import tpu as pltpu
from jax.experimental.pallas import tpu_sc as plsc

_CH = 24
_S = 8
_NL = 8


def _accum_chunk(buf, acc):
    def sbody(s, _):
        for l in range(_NL):
            sl = pl.ds(16 * l, 16)
            v = acc[s, sl]
            for cc in range(_CH):
                v = v + buf[cc, s, sl]
            acc[s, sl] = v
        return 0

    lax.fori_loop(0, _S, sbody, 0)


def _mul_chunk(inb, ob, maskb):
    def sbody(s, _):
        for l in range(_NL):
            sl = pl.ds(16 * l, 16)
            m = maskb[s, sl]
            for cc in range(_CH):
                ob[cc, s, sl] = inb[cc, s, sl] * m
        return 0

    lax.fori_loop(0, _S, sbody, 0)


def _sc_body(x_hbm, o_hbm, in0, in1, ob0, ob1, acc, keys, maskb, in_sems, out_sems, *, k, c_total):
    nch = c_total // _CH
    npair = nch // 2
    b = lax.axis_index("s") * 2 + lax.axis_index("c")

    ins = (in0, in1)
    obs = (ob0, ob1)

    def in_copy(q, slot):
        return pltpu.make_async_copy(
            x_hbm.at[b, pl.ds(q * _CH, _CH)], ins[slot], in_sems.at[slot]
        )

    def out_copy(q, slot):
        return pltpu.make_async_copy(
            obs[slot], o_hbm.at[b, pl.ds(q * _CH, _CH)], out_sems.at[slot]
        )

    def zbody(s, _):
        for l in range(_NL):
            acc[s, pl.ds(16 * l, 16)] = jnp.zeros((16,), jnp.float32)
        return 0

    lax.fori_loop(0, _S, zbody, 0)

    in_copy(0, 0).start()
    in_copy(1, 1).start()

    def p1body(p, _):
        q = 2 * p
        for slot in (0, 1):
            in_copy(q + slot, slot).wait()
            _accum_chunk(ins[slot], acc)

            @pl.when(p < npair - 1)
            def _():
                in_copy(q + slot + 2, slot).start()

        return 0

    lax.fori_loop(0, npair, p1body, 0)

    c31 = jnp.full((16,), 31, jnp.uint32)
    c1 = jnp.full((16,), 1, jnp.uint32)
    call1 = jnp.full((16,), 0xFFFFFFFF, jnp.uint32)
    csign = jnp.full((16,), 0x80000000, jnp.uint32)

    def kbody(s, _):
        for l in range(_NL):
            sl = pl.ds(16 * l, 16)
            bits = lax.bitcast_convert_type(acc[s, sl], jnp.uint32)
            neg = (bits >> c31) == c1
            keys[s, sl] = bits ^ jnp.where(neg, call1, csign)
        return 0

    lax.fori_loop(0, _S, kbody, 0)

    kk = jnp.int32(k)

    def splat_u32(x):
        return lax.broadcast_in_dim(x, (16,), ())

    def splat_i32(x):
        return lax.broadcast_in_dim(x, (16,), ())

    def count_where(pred):
        def sbody(s, cnt):
            for l in range(_NL):
                sl = pl.ds(16 * l, 16)
                kv = keys[s, sl]
                idx = lax.broadcasted_iota(jnp.int32, (16,), 0) + splat_i32(
                    s * 128 + 16 * l
                )
                cnt = cnt + jnp.sum(pred(kv, idx).astype(jnp.int32))
            return cnt

        return lax.fori_loop(0, _S, sbody, jnp.int32(0))

    def vround(i, t):
        shift = jnp.uint32(31) - i.astype(jnp.uint32)
        cand = t | (jnp.uint32(1) << shift)
        cand_v = splat_u32(cand)
        cnt = count_where(lambda kv, idx: kv >= cand_v)
        return jnp.where(cnt >= kk, cand, t)

    t = lax.fori_loop(0, 32, vround, jnp.uint32(0))
    t_v = splat_u32(t)

    cnt_gt = count_where(lambda kv, idx: kv > t_v)
    extra = kk - cnt_gt

    def iround(i, j):
        shift = jnp.int32(10) - i
        cand = j | (jnp.int32(1) << shift)
        cand_v = splat_i32(cand)
        cnt = count_where(lambda kv, idx: (kv == t_v) & (idx < cand_v))
        return jnp.where(cnt <= extra, cand, j)

    jmax = lax.fori_loop(0, 11, iround, jnp.int32(0))
    jmax_v = splat_i32(jmax)

    one_f = jnp.full((16,), 1.0, jnp.float32)
    zero_f = jnp.full((16,), 0.0, jnp.float32)

    def mbody(s, _):
        for l in range(_NL):
            sl = pl.ds(16 * l, 16)
            kv = keys[s, sl]
            idx = lax.broadcasted_iota(jnp.int32, (16,), 0) + splat_i32(
                s * 128 + 16 * l
            )
            keep = (kv > t_v) | ((kv == t_v) & (idx < jmax_v))
            maskb[s, sl] = jnp.where(keep, one_f, zero_f)
        return 0

    lax.fori_loop(0, _S, mbody, 0)

    in_copy(0, 0).start()
    in_copy(1, 1).start()

    def p2body(p, _):
        q = 2 * p
        for slot in (0, 1):
            in_copy(q + slot, slot).wait()

            @pl.when(p > 0)
            def _():
                out_copy(q + slot - 2, slot).wait()

            _mul_chunk(ins[slot], obs[slot], maskb)
            out_copy(q + slot, slot).start()

            @pl.when(p < npair - 1)
            def _():
                in_copy(q + slot + 2, slot).start()

        return 0

    lax.fori_loop(0, npair, p2body, 0)

    out_copy(nch - 2, 0).wait()
    out_copy(nch - 1, 1).wait()


def kernel(x):
    B, C, H, W = x.shape
    HW = H * W
    k = int(HW * 0.5)
    assert HW == _S * 128 and C % (2 * _CH) == 0
    xr = x.reshape(B, C, _S, 128)

    mesh = plsc.VectorSubcoreMesh(core_axis_name="c", subcore_axis_name="s")

    body = functools.partial(_sc_body, k=k, c_total=C)
    f = pl.kernel(
        body,
        out_type=jax.ShapeDtypeStruct((B, C, _S, 128), jnp.float32),
        mesh=mesh,
        compiler_params=pltpu.CompilerParams(needs_layout_passes=False),
        scratch_types=[
            pltpu.VMEM((_CH, _S, 128), jnp.float32),
            pltpu.VMEM((_CH, _S, 128), jnp.float32),
            pltpu.VMEM((_CH, _S, 128), jnp.float32),
            pltpu.VMEM((_CH, _S, 128), jnp.float32),
            pltpu.VMEM((_S, 128), jnp.float32),
            pltpu.VMEM((_S, 128), jnp.uint32),
            pltpu.VMEM((_S, 128), jnp.float32),
            pltpu.SemaphoreType.DMA((2,)),
            pltpu.SemaphoreType.DMA((2,)),
        ],
    )
    out = f(xr)
    return out.reshape(B, C, H, W)
